# Initial kernel scaffold; baseline (speedup 1.0000x reference)
import jax
import jax.numpy as jnp
from jax.experimental import pallas as pl


def kernel(u_mem, source_rotor, target_rotor, W_g, W1, b1, W2, b2, Wd1, bd1, Wd2, bd2, cayley, rev_signs):
    raise NotImplementedError("write your pallas kernel here")


# speedup vs baseline: 1.6058x; 1.6058x over previous
import functools

import jax
import jax.numpy as jnp
import numpy as np
from jax.experimental import pallas as pl
from jax.experimental.pallas import tpu as pltpu

NUM_EXPERTS = 16
DIM = 64
EXPERT_DIM = 128
OUTPUT_DIM = 256
HID = NUM_EXPERTS * EXPERT_DIM

HIGH = jax.lax.Precision.HIGHEST


def _rotor_matrix_kernel(sr_ref, tr_ref, rev_ref, c_ikj_ref, c_jki_ref, m_ref):
    sr = sr_ref[:]
    tr = tr_ref[:]
    rev = rev_ref[:]
    rs = sr / (jnp.sqrt(jnp.sum(sr * sr)) + 1e-8)
    rt = tr / (jnp.sqrt(jnp.sum(tr * tr)) + 1e-8)
    rs_rev = rs * rev
    rt_rev = rt * rev

    c_ikj = c_ikj_ref[:]
    c_jki = c_jki_ref[:]

    def contract(c, v):
        return jnp.sum(c * v.reshape(1, 1, DIM), axis=2)

    a1 = contract(c_ikj, rs)
    b1 = contract(c_jki, rs_rev)
    a2 = contract(c_ikj, rt_rev)
    b2 = contract(c_jki, rt)
    m = jnp.dot(jnp.dot(jnp.dot(a1, b1, precision=HIGH), a2, precision=HIGH),
                b2, precision=HIGH)
    m_ref[:] = m


def _main_kernel(x_ref, wg_ref, w1_ref, b1_ref, w2_ref, b2_ref,
                 exp_ref, m_ref, wd1_ref, bd1_ref, wd2_ref, bd2_ref,
                 out_ref, route_ref, g_ref, probs_ref):
    x = x_ref[:]
    logits = jnp.dot(x, wg_ref[:])
    lmax = jnp.max(logits, axis=1, keepdims=True)
    ex = jnp.exp(logits - lmax)
    probs = ex / jnp.sum(ex, axis=1, keepdims=True)
    probs_ref[:] = probs

    m1 = jnp.max(probs, axis=1, keepdims=True)
    idx = jax.lax.broadcasted_iota(jnp.int32, probs.shape, 1)
    first = jnp.min(jnp.where(probs == m1, idx, NUM_EXPERTS), axis=1,
                    keepdims=True)
    p_wo = jnp.where(idx == first, -1.0, probs)
    m2 = jnp.max(p_wo, axis=1, keepdims=True)
    w = jnp.where(probs >= m2, probs, 0.0) / (m1 + m2)

    h = jnp.dot(x, w1_ref[:]) + b1_ref[:]
    h = jax.nn.gelu(h)
    w_exp = jnp.dot(w, exp_ref[:])
    route = (jnp.dot(h * w_exp, w2_ref[:])
             + jnp.dot(w, b2_ref[:]))
    route_ref[:] = route

    g = jnp.dot(route, m_ref[:])
    g_ref[:] = g

    hd = jax.nn.gelu(jnp.dot(g, wd1_ref[:]) + bd1_ref[:])
    out_ref[:] = jnp.dot(hd, wd2_ref[:]) + bd2_ref[:]


@functools.partial(jax.jit, static_argnames=("interpret",))
def kernel(u_mem, source_rotor, target_rotor, W_g, W1, b1, W2, b2,
           Wd1, bd1, Wd2, bd2, cayley, rev_signs, interpret=False):
    B = u_mem.shape[0]
    BT = 512
    nb = B // BT

    c_ikj = cayley.transpose(0, 2, 1)
    c_jki = cayley.transpose(1, 2, 0)
    m = pl.pallas_call(
        _rotor_matrix_kernel,
        out_shape=jax.ShapeDtypeStruct((DIM, DIM), jnp.float32),
        interpret=interpret,
    )(source_rotor.reshape(1, DIM), target_rotor.reshape(1, DIM),
      rev_signs.reshape(1, DIM), c_ikj, c_jki)

    w1_flat = W1.transpose(1, 0, 2).reshape(DIM, HID)
    b1_flat = b1.reshape(1, HID)
    w2_flat = W2.reshape(HID, DIM)
    expand = jnp.asarray(np.kron(np.eye(NUM_EXPERTS, dtype=np.float32),
                                 np.ones((1, EXPERT_DIM), np.float32)))

    tok = lambda i: (i, 0)
    rep = lambda i: (0, 0)
    out, route, g, probs = pl.pallas_call(
        _main_kernel,
        grid=(nb,),
        in_specs=[
            pl.BlockSpec((BT, DIM), tok),
            pl.BlockSpec((DIM, NUM_EXPERTS), rep),
            pl.BlockSpec((DIM, HID), rep),
            pl.BlockSpec((1, HID), rep),
            pl.BlockSpec((HID, DIM), rep),
            pl.BlockSpec((NUM_EXPERTS, DIM), rep),
            pl.BlockSpec((NUM_EXPERTS, HID), rep),
            pl.BlockSpec((DIM, DIM), rep),
            pl.BlockSpec((DIM, EXPERT_DIM), rep),
            pl.BlockSpec((1, EXPERT_DIM), rep),
            pl.BlockSpec((EXPERT_DIM, OUTPUT_DIM), rep),
            pl.BlockSpec((1, OUTPUT_DIM), rep),
        ],
        out_specs=[
            pl.BlockSpec((BT, OUTPUT_DIM), tok),
            pl.BlockSpec((BT, DIM), tok),
            pl.BlockSpec((BT, DIM), tok),
            pl.BlockSpec((BT, NUM_EXPERTS), tok),
        ],
        out_shape=[
            jax.ShapeDtypeStruct((B, OUTPUT_DIM), jnp.float32),
            jax.ShapeDtypeStruct((B, DIM), jnp.float32),
            jax.ShapeDtypeStruct((B, DIM), jnp.float32),
            jax.ShapeDtypeStruct((B, NUM_EXPERTS), jnp.float32),
        ],
        compiler_params=pltpu.CompilerParams(
            dimension_semantics=("parallel",),
        ),
        interpret=interpret,
    )(u_mem, W_g, w1_flat, b1_flat, w2_flat, b2,
      expand, m, Wd1, bd1.reshape(1, EXPERT_DIM), Wd2,
      bd2.reshape(1, OUTPUT_DIM))
    return (out, route, g, probs)
